# baseline (speedup 1.0000x reference)
import functools

import jax
import jax.numpy as jnp
from jax import lax
from jax.experimental import pallas as pl
from jax.experimental.pallas import tpu as pltpu
from jax.experimental.pallas import tpu_sc as plsc

N = 10000
E = 320000
F = 128
NC = 10

BN = 1000
NBLK = N // BN

NSUB = 16
EDGES_PER_SUB = E // NSUB
CHUNK = 80
NCHUNK = EDGES_PER_SUB // CHUNK
INIT_ROWS = 640
WB = 80
NBUF = 4
G = 10
NGROUPS = NCHUNK // G
SD = 2 * G
LOOK = 2


def _lrelu(v):
    return jnp.where(v >= 0, v, 0.2 * v)


def _layer1_body(x_ref, me_ref, wt_ref, scale_ref, bias_ref, y_ref, base_ref):
    xb = x_ref[0]
    xb = jnp.where(xb == 0.0, me_ref[0], xb)
    y = jnp.dot(xb, wt_ref[0], preferred_element_type=jnp.float32)
    y_ref[0] = y
    base_ref[0] = y * scale_ref[0] + bias_ref[0]


def _layer2_body(a_ref, wt_ref, scale_ref, bias_ref, y_ref, base_ref):
    h = _lrelu(a_ref[0])
    y = jnp.dot(h, wt_ref[0], preferred_element_type=jnp.float32)
    y_ref[0] = y
    base_ref[0] = y * scale_ref[0] + bias_ref[0]


def _branch_specs():
    row = pl.BlockSpec((1, BN, F), lambda b, i: (b, i, 0))
    per_branch = pl.BlockSpec((1, 1, F), lambda b, i: (b, 0, 0))
    w = pl.BlockSpec((1, F, F), lambda b, i: (b, 0, 0))
    return row, per_branch, w


def _layer1_call(X, ME, WT, SCALE, BIAS):
    row, per_branch, w = _branch_specs()
    return pl.pallas_call(
        _layer1_body,
        grid=(2, NBLK),
        in_specs=[row, per_branch, w, per_branch, per_branch],
        out_specs=[row, row],
        out_shape=[jax.ShapeDtypeStruct((2, N, F), jnp.float32)] * 2,
    )(X, ME, WT, SCALE, BIAS)


def _layer2_call(A, WT, SCALE, BIAS):
    row, per_branch, w = _branch_specs()
    return pl.pallas_call(
        _layer2_body,
        grid=(2, NBLK),
        in_specs=[row, w, per_branch, per_branch],
        out_specs=[row, row],
        out_shape=[jax.ShapeDtypeStruct((2, N, F), jnp.float32)] * 2,
    )(A, WT, SCALE, BIAS)


def _head_body(a_ref, mask_ref, wmt_ref, bm_ref, w1t_ref, b1_ref, w2t_ref,
               b2_ref, xm_ref, lg_ref):
    he = _lrelu(a_ref[0])
    hc = _lrelu(a_ref[1])
    wm = wmt_ref[...]
    xm = (jnp.dot(he, wm[:F], preferred_element_type=jnp.float32)
          + jnp.dot(hc, wm[F:], preferred_element_type=jnp.float32)
          + bm_ref[...])
    xm = _lrelu(xm)
    xm_ref[...] = xm
    central = xm * mask_ref[...][:, :1]
    h = jnp.maximum(
        jnp.dot(central, w1t_ref[...], preferred_element_type=jnp.float32)
        + b1_ref[...], 0.0)
    lg_ref[...] = (jnp.dot(h, w2t_ref[...], preferred_element_type=jnp.float32)
                   + b2_ref[...])


def _head_call(A2, MASK, WMT, BM, W1T, B1, W2T, B2):
    whole = lambda shape: pl.BlockSpec(shape, lambda i: tuple(0 for _ in shape))
    return pl.pallas_call(
        _head_body,
        grid=(NBLK,),
        in_specs=[
            pl.BlockSpec((2, BN, F), lambda i: (0, i, 0)),
            pl.BlockSpec((BN, F), lambda i: (i, 0)),
            whole((2 * F, NC)),
            whole((1, NC)),
            whole((NC, F)),
            whole((1, F)),
            whole((F, NC)),
            whole((1, NC)),
        ],
        out_specs=[
            pl.BlockSpec((BN, NC), lambda i: (i, 0)),
            pl.BlockSpec((BN, NC), lambda i: (i, 0)),
        ],
        out_shape=[jax.ShapeDtypeStruct((N, NC), jnp.float32)] * 2,
    )(A2, MASK, WMT, BM, W1T, B1, W2T, B2)


def _sc_body(y_hbm, base_hbm, sd_hbm, out_hbm,
             idx2, rows_v, acc_sh, gsem, isem, ssem):
    cid = lax.axis_index("c")
    sid = lax.axis_index("s")

    rows0 = pl.multiple_of(jnp.minimum(sid * INIT_ROWS, N - INIT_ROWS), 8)
    pltpu.sync_copy(base_hbm.at[pl.ds(cid * N + rows0, INIT_ROWS)],
                    acc_sh.at[pl.ds(rows0, INIT_ROWS)])

    pltpu.sync_copy(sd_hbm.at[cid, sid, 0], idx2.at[0])
    plsc.subcore_barrier()

    for b in range(LOOK):
        pltpu.async_copy(y_hbm.at[idx2.at[0, 2 * b]], rows_v.at[b],
                         gsem.at[b])

    def body(i, carry):
        p = lax.rem(i, NBUF)
        g = lax.div(i, G)
        j = lax.rem(i, G)
        gp = lax.rem(g, 2)

        @pl.when(jnp.logical_and(j == 2, g + 1 < NGROUPS))
        def _():
            pltpu.async_copy(sd_hbm.at[cid, sid, g + 1],
                             idx2.at[1 - gp], isem.at[1 - gp])

        pltpu.make_async_copy(y_hbm.at[idx2.at[0, 0]], rows_v.at[p],
                              gsem.at[p]).wait()
        pltpu.async_copy(rows_v.at[p], acc_sh.at[idx2.at[gp, 2 * j + 1]],
                         ssem.at[p], add=True)

        @pl.when(jnp.logical_and(j == G - LOOK, g + 1 < NGROUPS))
        def _():
            pltpu.make_async_copy(sd_hbm.at[cid, sid, 0], idx2.at[1 - gp],
                                  isem.at[1 - gp]).wait()

        nxt = i + LOOK
        q = lax.rem(nxt, NBUF)

        @pl.when(nxt >= NBUF)
        def _():
            pltpu.make_async_copy(rows_v.at[0], acc_sh.at[idx2.at[0, 1]],
                                  ssem.at[q]).wait()

        @pl.when(nxt < NCHUNK)
        def _():
            gn = lax.rem(lax.div(nxt, G), 2)
            jn = lax.rem(nxt, G)
            pltpu.async_copy(y_hbm.at[idx2.at[gn, 2 * jn]], rows_v.at[q],
                             gsem.at[q])
        return carry

    lax.fori_loop(0, NCHUNK, body, 0)

    for b in range(NBUF - LOOK):
        q = (NCHUNK - 1 - b) % NBUF
        pltpu.make_async_copy(rows_v.at[0], acc_sh.at[idx2.at[0, 1]],
                              ssem.at[q]).wait()
    plsc.subcore_barrier()

    for j in range(INIT_ROWS // WB):
        r = pl.multiple_of(rows0 + j * WB, 8)
        pltpu.sync_copy(acc_sh.at[pl.ds(r, WB)], rows_v.at[0])
        pltpu.sync_copy(rows_v.at[0], out_hbm.at[pl.ds(cid * N + r, WB)])


def _segsum_call(y_flat, base_flat, sd_r):
    mesh = plsc.VectorSubcoreMesh(core_axis_name="c", subcore_axis_name="s")
    k = pl.kernel(
        _sc_body,
        out_type=jax.ShapeDtypeStruct((2 * N, F), jnp.float32),
        mesh=mesh,
        scratch_types=[
            pltpu.VMEM((2, SD, CHUNK), jnp.int32),
            pltpu.VMEM((NBUF, WB, F), jnp.float32),
            pltpu.VMEM_SHARED((N, F), jnp.float32),
            pltpu.SemaphoreType.DMA((NBUF,)),
            pltpu.SemaphoreType.DMA((2,)),
            pltpu.SemaphoreType.DMA((NBUF,)),
        ],
    )
    return k(y_flat, base_flat, sd_r)


def kernel(x, c, edge_index, central_node_index, me_x, me_c,
           eps1e, W1e, b1e, eps2e, W2e, b2e,
           eps1c, W1c, b1c, eps2c, W2c, b2c,
           Wm, bm, Wmlp1, bmlp1, Wmlp2, bmlp2):
    X = jnp.stack([x, c])
    ME = jnp.stack([me_x, me_c]).reshape(2, 1, F)
    WT1 = jnp.stack([W1e.T, W1c.T])
    S1 = jnp.stack([jnp.full((1, F), 2.0 + eps1e, jnp.float32),
                    jnp.full((1, F), 2.0 + eps1c, jnp.float32)])
    B1 = jnp.stack([b1e, b1c]).reshape(2, 1, F)
    WT2 = jnp.stack([W2e.T, W2c.T])
    S2 = jnp.stack([jnp.full((1, F), 2.0 + eps2e, jnp.float32),
                    jnp.full((1, F), 2.0 + eps2c, jnp.float32)])
    B2 = jnp.stack([b2e, b2c]).reshape(2, 1, F)

    src = edge_index[0]
    dst = edge_index[1]
    src_b = jnp.stack([src, src + N]).reshape(2, NSUB, NGROUPS, G, CHUNK)
    dst_b = jnp.broadcast_to(dst.reshape(1, NSUB, NGROUPS, G, CHUNK),
                             src_b.shape)
    sd_r = jnp.stack([src_b, dst_b], axis=4).reshape(
        2, NSUB, NGROUPS, SD, CHUNK)
    y1, base1 = _layer1_call(X, ME, WT1, S1, B1)
    acc1 = _segsum_call(y1.reshape(2 * N, F), base1.reshape(2 * N, F), sd_r)
    y2, base2 = _layer2_call(acc1.reshape(2, N, F), WT2, S2, B2)
    acc2 = _segsum_call(y2.reshape(2 * N, F), base2.reshape(2 * N, F), sd_r)

    maskf = jnp.broadcast_to(
        (central_node_index == 1).astype(jnp.float32)[:, None], (N, F))
    xm, logits = _head_call(acc2.reshape(2, N, F), maskf, Wm.T,
                            bm.reshape(1, NC), Wmlp1.T, bmlp1.reshape(1, F),
                            Wmlp2.T, bmlp2.reshape(1, NC))
    return (xm, logits)


# speedup vs baseline: 1.1866x; 1.1866x over previous
import functools

import jax
import jax.numpy as jnp
from jax import lax
from jax.experimental import pallas as pl
from jax.experimental.pallas import tpu as pltpu
from jax.experimental.pallas import tpu_sc as plsc

N = 10000
E = 320000
F = 128
NC = 10

BN = 1000
NBLK = N // BN

NSUB = 16
EDGES_PER_SUB = E // NSUB
CHUNK = 80
NCHUNK = EDGES_PER_SUB // CHUNK
INIT_ROWS = 640
WB = 80
NBUF = 4
G = 10
NGROUPS = NCHUNK // G
SD = 2 * G
LOOK = 2


def _lrelu(v):
    return jnp.where(v >= 0, v, 0.2 * v)


def _layer1_body(x_ref, me_ref, wt_ref, scale_ref, bias_ref, y_ref, base_ref):
    xb = x_ref[0]
    xb = jnp.where(xb == 0.0, me_ref[0], xb)
    y = jnp.dot(xb, wt_ref[0], preferred_element_type=jnp.float32)
    y_ref[0] = y
    base_ref[0] = y * scale_ref[0] + bias_ref[0]


def _layer2_body(a_ref, wt_ref, scale_ref, bias_ref, y_ref, base_ref):
    h = _lrelu(a_ref[0])
    y = jnp.dot(h, wt_ref[0], preferred_element_type=jnp.float32)
    y_ref[0] = y
    base_ref[0] = y * scale_ref[0] + bias_ref[0]


def _branch_specs():
    row = pl.BlockSpec((1, BN, F), lambda b, i: (b, i, 0))
    per_branch = pl.BlockSpec((1, 1, F), lambda b, i: (b, 0, 0))
    w = pl.BlockSpec((1, F, F), lambda b, i: (b, 0, 0))
    return row, per_branch, w


def _layer1_call(X, ME, WT, SCALE, BIAS):
    row, per_branch, w = _branch_specs()
    return pl.pallas_call(
        _layer1_body,
        grid=(2, NBLK),
        in_specs=[row, per_branch, w, per_branch, per_branch],
        out_specs=[row, row],
        out_shape=[jax.ShapeDtypeStruct((2, N, F), jnp.float32)] * 2,
    )(X, ME, WT, SCALE, BIAS)


def _layer2_call(A, WT, SCALE, BIAS):
    row, per_branch, w = _branch_specs()
    return pl.pallas_call(
        _layer2_body,
        grid=(2, NBLK),
        in_specs=[row, w, per_branch, per_branch],
        out_specs=[row, row],
        out_shape=[jax.ShapeDtypeStruct((2, N, F), jnp.float32)] * 2,
    )(A, WT, SCALE, BIAS)


def _head_body(a_ref, mask_ref, wmt_ref, bm_ref, w1t_ref, b1_ref, w2t_ref,
               b2_ref, xm_ref, lg_ref):
    he = _lrelu(a_ref[0])
    hc = _lrelu(a_ref[1])
    wm = wmt_ref[...]
    xm = (jnp.dot(he, wm[:F], preferred_element_type=jnp.float32)
          + jnp.dot(hc, wm[F:], preferred_element_type=jnp.float32)
          + bm_ref[...])
    xm = _lrelu(xm)
    xm_ref[...] = xm
    central = xm * mask_ref[...][:, :1]
    h = jnp.maximum(
        jnp.dot(central, w1t_ref[...], preferred_element_type=jnp.float32)
        + b1_ref[...], 0.0)
    lg_ref[...] = (jnp.dot(h, w2t_ref[...], preferred_element_type=jnp.float32)
                   + b2_ref[...])


def _head_call(A2, MASK, WMT, BM, W1T, B1, W2T, B2):
    whole = lambda shape: pl.BlockSpec(shape, lambda i: tuple(0 for _ in shape))
    return pl.pallas_call(
        _head_body,
        grid=(NBLK,),
        in_specs=[
            pl.BlockSpec((2, BN, F), lambda i: (0, i, 0)),
            pl.BlockSpec((BN, F), lambda i: (i, 0)),
            whole((2 * F, NC)),
            whole((1, NC)),
            whole((NC, F)),
            whole((1, F)),
            whole((F, NC)),
            whole((1, NC)),
        ],
        out_specs=[
            pl.BlockSpec((BN, NC), lambda i: (i, 0)),
            pl.BlockSpec((BN, NC), lambda i: (i, 0)),
        ],
        out_shape=[jax.ShapeDtypeStruct((N, NC), jnp.float32)] * 2,
    )(A2, MASK, WMT, BM, W1T, B1, W2T, B2)


def _sc_body(y_hbm, base_hbm, sd_hbm, out_hbm,
             idx2, rows_v, acc_sh, gsem, isem):
    cid = lax.axis_index("c")
    sid = lax.axis_index("s")

    rows0 = pl.multiple_of(jnp.minimum(sid * INIT_ROWS, N - INIT_ROWS), 8)
    pltpu.sync_copy(base_hbm.at[pl.ds(cid * N + rows0, INIT_ROWS)],
                    acc_sh.at[pl.ds(rows0, INIT_ROWS)])

    pltpu.sync_copy(sd_hbm.at[cid, sid, 0], idx2.at[0])
    plsc.subcore_barrier()

    for b in range(NBUF):
        pltpu.async_copy(y_hbm.at[idx2.at[0, 2 * b]], rows_v.at[b],
                         gsem.at[b])

    def body(i, carry):
        p = lax.rem(i, NBUF)
        g = lax.div(i, G)
        j = lax.rem(i, G)
        gp = lax.rem(g, 2)

        @pl.when(jnp.logical_and(j == 0, g + 1 < NGROUPS))
        def _():
            pltpu.async_copy(sd_hbm.at[cid, sid, g + 1],
                             idx2.at[1 - gp], isem.at[1 - gp])

        pltpu.make_async_copy(y_hbm.at[idx2.at[0, 0]], rows_v.at[p],
                              gsem.at[p]).wait()
        pltpu.sync_copy(rows_v.at[p], acc_sh.at[idx2.at[gp, 2 * j + 1]],
                        add=True)

        @pl.when(jnp.logical_and(j == G - NBUF, g + 1 < NGROUPS))
        def _():
            pltpu.make_async_copy(sd_hbm.at[cid, sid, 0], idx2.at[1 - gp],
                                  isem.at[1 - gp]).wait()

        nxt = i + NBUF

        @pl.when(nxt < NCHUNK)
        def _():
            gn = lax.rem(lax.div(nxt, G), 2)
            jn = lax.rem(nxt, G)
            pltpu.async_copy(y_hbm.at[idx2.at[gn, 2 * jn]], rows_v.at[p],
                             gsem.at[p])
        return carry

    lax.fori_loop(0, NCHUNK, body, 0)
    plsc.subcore_barrier()

    for j in range(INIT_ROWS // WB):
        r = pl.multiple_of(rows0 + j * WB, 8)
        pltpu.sync_copy(acc_sh.at[pl.ds(r, WB)], rows_v.at[0])
        pltpu.sync_copy(rows_v.at[0], out_hbm.at[pl.ds(cid * N + r, WB)])


def _segsum_call(y_flat, base_flat, sd_r):
    mesh = plsc.VectorSubcoreMesh(core_axis_name="c", subcore_axis_name="s")
    k = pl.kernel(
        _sc_body,
        out_type=jax.ShapeDtypeStruct((2 * N, F), jnp.float32),
        mesh=mesh,
        scratch_types=[
            pltpu.VMEM((2, SD, CHUNK), jnp.int32),
            pltpu.VMEM((NBUF, WB, F), jnp.float32),
            pltpu.VMEM_SHARED((N, F), jnp.float32),
            pltpu.SemaphoreType.DMA((NBUF,)),
            pltpu.SemaphoreType.DMA((2,)),
        ],
    )
    return k(y_flat, base_flat, sd_r)


def kernel(x, c, edge_index, central_node_index, me_x, me_c,
           eps1e, W1e, b1e, eps2e, W2e, b2e,
           eps1c, W1c, b1c, eps2c, W2c, b2c,
           Wm, bm, Wmlp1, bmlp1, Wmlp2, bmlp2):
    X = jnp.stack([x, c])
    ME = jnp.stack([me_x, me_c]).reshape(2, 1, F)
    WT1 = jnp.stack([W1e.T, W1c.T])
    S1 = jnp.stack([jnp.full((1, F), 2.0 + eps1e, jnp.float32),
                    jnp.full((1, F), 2.0 + eps1c, jnp.float32)])
    B1 = jnp.stack([b1e, b1c]).reshape(2, 1, F)
    WT2 = jnp.stack([W2e.T, W2c.T])
    S2 = jnp.stack([jnp.full((1, F), 2.0 + eps2e, jnp.float32),
                    jnp.full((1, F), 2.0 + eps2c, jnp.float32)])
    B2 = jnp.stack([b2e, b2c]).reshape(2, 1, F)

    src = edge_index[0]
    dst = edge_index[1]
    src_b = jnp.stack([src, src + N]).reshape(2, NSUB, NGROUPS, G, CHUNK)
    dst_b = jnp.broadcast_to(dst.reshape(1, NSUB, NGROUPS, G, CHUNK),
                             src_b.shape)
    sd_r = jnp.stack([src_b, dst_b], axis=4).reshape(
        2, NSUB, NGROUPS, SD, CHUNK)
    y1, base1 = _layer1_call(X, ME, WT1, S1, B1)
    acc1 = _segsum_call(y1.reshape(2 * N, F), base1.reshape(2 * N, F), sd_r)
    y2, base2 = _layer2_call(acc1.reshape(2, N, F), WT2, S2, B2)
    acc2 = _segsum_call(y2.reshape(2 * N, F), base2.reshape(2 * N, F), sd_r)

    maskf = jnp.broadcast_to(
        (central_node_index == 1).astype(jnp.float32)[:, None], (N, F))
    xm, logits = _head_call(acc2.reshape(2, N, F), maskf, Wm.T,
                            bm.reshape(1, NC), Wmlp1.T, bmlp1.reshape(1, F),
                            Wmlp2.T, bmlp2.reshape(1, NC))
    return (xm, logits)


# speedup vs baseline: 1.1930x; 1.0054x over previous
import functools

import jax
import jax.numpy as jnp
from jax import lax
from jax.experimental import pallas as pl
from jax.experimental.pallas import tpu as pltpu
from jax.experimental.pallas import tpu_sc as plsc

N = 10000
E = 320000
F = 128
NC = 10

BN = 1000
NBLK = N // BN

NSUB = 16
EDGES_PER_SUB = E // NSUB
CHUNK = 80
NCHUNK = EDGES_PER_SUB // CHUNK
INIT_ROWS = 640
WB = 80
NBUF = 4
G = 10
NGROUPS = NCHUNK // G
SD = 2 * G
LOOK = 2


def _lrelu(v):
    return jnp.where(v >= 0, v, 0.2 * v)


def _layer1_body(x_ref, me_ref, wt_ref, scale_ref, bias_ref, y_ref, base_ref):
    xb = x_ref[0]
    xb = jnp.where(xb == 0.0, me_ref[0], xb)
    y = jnp.dot(xb, wt_ref[0], preferred_element_type=jnp.float32)
    y_ref[0] = y
    base_ref[0] = y * scale_ref[0] + bias_ref[0]


def _layer2_body(a_ref, wt_ref, scale_ref, bias_ref, y_ref, base_ref):
    h = _lrelu(a_ref[0])
    y = jnp.dot(h, wt_ref[0], preferred_element_type=jnp.float32)
    y_ref[0] = y
    base_ref[0] = y * scale_ref[0] + bias_ref[0]


def _branch_specs():
    row = pl.BlockSpec((1, BN, F), lambda b, i: (b, i, 0))
    per_branch = pl.BlockSpec((1, 1, F), lambda b, i: (b, 0, 0))
    w = pl.BlockSpec((1, F, F), lambda b, i: (b, 0, 0))
    return row, per_branch, w


def _layer1_call(X, ME, WT, SCALE, BIAS):
    row, per_branch, w = _branch_specs()
    return pl.pallas_call(
        _layer1_body,
        grid=(2, NBLK),
        in_specs=[row, per_branch, w, per_branch, per_branch],
        out_specs=[row, row],
        out_shape=[jax.ShapeDtypeStruct((2, N, F), jnp.float32)] * 2,
    )(X, ME, WT, SCALE, BIAS)


def _layer2_call(A, WT, SCALE, BIAS):
    row, per_branch, w = _branch_specs()
    return pl.pallas_call(
        _layer2_body,
        grid=(2, NBLK),
        in_specs=[row, w, per_branch, per_branch],
        out_specs=[row, row],
        out_shape=[jax.ShapeDtypeStruct((2, N, F), jnp.float32)] * 2,
    )(A, WT, SCALE, BIAS)


def _head_body(a_ref, mask_ref, wmt_ref, bm_ref, w1t_ref, b1_ref, w2t_ref,
               b2_ref, xm_ref, lg_ref):
    he = _lrelu(a_ref[0])
    hc = _lrelu(a_ref[1])
    wm = wmt_ref[...]
    xm = (jnp.dot(he, wm[:F], preferred_element_type=jnp.float32)
          + jnp.dot(hc, wm[F:], preferred_element_type=jnp.float32)
          + bm_ref[...])
    xm = _lrelu(xm)
    xm_ref[...] = xm
    central = xm * mask_ref[...][:, :1]
    h = jnp.maximum(
        jnp.dot(central, w1t_ref[...], preferred_element_type=jnp.float32)
        + b1_ref[...], 0.0)
    lg_ref[...] = (jnp.dot(h, w2t_ref[...], preferred_element_type=jnp.float32)
                   + b2_ref[...])


def _head_call(A2, MASK, WMT, BM, W1T, B1, W2T, B2):
    whole = lambda shape: pl.BlockSpec(shape, lambda i: tuple(0 for _ in shape))
    return pl.pallas_call(
        _head_body,
        grid=(NBLK,),
        in_specs=[
            pl.BlockSpec((2, BN, F), lambda i: (0, i, 0)),
            pl.BlockSpec((BN, 8), lambda i: (i, 0)),
            whole((2 * F, NC)),
            whole((1, NC)),
            whole((NC, F)),
            whole((1, F)),
            whole((F, NC)),
            whole((1, NC)),
        ],
        out_specs=[
            pl.BlockSpec((BN, NC), lambda i: (i, 0)),
            pl.BlockSpec((BN, NC), lambda i: (i, 0)),
        ],
        out_shape=[jax.ShapeDtypeStruct((N, NC), jnp.float32)] * 2,
    )(A2, MASK, WMT, BM, W1T, B1, W2T, B2)


def _sc_body(y_hbm, base_hbm, sd_hbm, out_hbm,
             idx2, rows_v, acc_sh, gsem, isem):
    cid = lax.axis_index("c")
    sid = lax.axis_index("s")

    rows0 = pl.multiple_of(jnp.minimum(sid * INIT_ROWS, N - INIT_ROWS), 8)
    pltpu.sync_copy(base_hbm.at[pl.ds(cid * N + rows0, INIT_ROWS)],
                    acc_sh.at[pl.ds(rows0, INIT_ROWS)])

    pltpu.sync_copy(sd_hbm.at[cid, sid, 0], idx2.at[0])
    plsc.subcore_barrier()

    for b in range(NBUF):
        pltpu.async_copy(y_hbm.at[idx2.at[0, 2 * b]],
                         rows_v.at[pl.ds(b * CHUNK, CHUNK)], gsem.at[b])

    def body(i, carry):
        p = lax.rem(i, NBUF)
        g = lax.div(i, G)
        j = lax.rem(i, G)
        gp = lax.rem(g, 2)

        @pl.when(jnp.logical_and(j == 0, g + 1 < NGROUPS))
        def _():
            pltpu.async_copy(sd_hbm.at[cid, sid, g + 1],
                             idx2.at[1 - gp], isem.at[1 - gp])

        rb = rows_v.at[pl.ds(pl.multiple_of(p * CHUNK, 8), CHUNK)]
        pltpu.make_async_copy(y_hbm.at[idx2.at[0, 0]], rb,
                              gsem.at[p]).wait()
        pltpu.sync_copy(rb, acc_sh.at[idx2.at[gp, 2 * j + 1]], add=True)

        @pl.when(jnp.logical_and(j == G - NBUF, g + 1 < NGROUPS))
        def _():
            pltpu.make_async_copy(sd_hbm.at[cid, sid, 0], idx2.at[1 - gp],
                                  isem.at[1 - gp]).wait()

        nxt = i + NBUF

        @pl.when(nxt < NCHUNK)
        def _():
            gn = lax.rem(lax.div(nxt, G), 2)
            jn = lax.rem(nxt, G)
            pltpu.async_copy(y_hbm.at[idx2.at[gn, 2 * jn]], rb, gsem.at[p])
        return carry

    lax.fori_loop(0, NCHUNK, body, 0)
    plsc.subcore_barrier()

    for j in range(INIT_ROWS // (NBUF * WB)):
        r = pl.multiple_of(rows0 + j * NBUF * WB, 8)
        pltpu.sync_copy(acc_sh.at[pl.ds(r, NBUF * WB)], rows_v)
        pltpu.sync_copy(rows_v, out_hbm.at[pl.ds(cid * N + r, NBUF * WB)])


def _segsum_call(y_flat, base_flat, sd_r):
    mesh = plsc.VectorSubcoreMesh(core_axis_name="c", subcore_axis_name="s")
    k = pl.kernel(
        _sc_body,
        out_type=jax.ShapeDtypeStruct((2 * N, F), jnp.float32),
        mesh=mesh,
        scratch_types=[
            pltpu.VMEM((2, SD, CHUNK), jnp.int32),
            pltpu.VMEM((NBUF * WB, F), jnp.float32),
            pltpu.VMEM_SHARED((N, F), jnp.float32),
            pltpu.SemaphoreType.DMA((NBUF,)),
            pltpu.SemaphoreType.DMA((2,)),
        ],
    )
    return k(y_flat, base_flat, sd_r)


def kernel(x, c, edge_index, central_node_index, me_x, me_c,
           eps1e, W1e, b1e, eps2e, W2e, b2e,
           eps1c, W1c, b1c, eps2c, W2c, b2c,
           Wm, bm, Wmlp1, bmlp1, Wmlp2, bmlp2):
    X = jnp.stack([x, c])
    ME = jnp.stack([me_x, me_c]).reshape(2, 1, F)
    WT1 = jnp.stack([W1e.T, W1c.T])
    S1 = jnp.stack([jnp.full((1, F), 2.0 + eps1e, jnp.float32),
                    jnp.full((1, F), 2.0 + eps1c, jnp.float32)])
    B1 = jnp.stack([b1e, b1c]).reshape(2, 1, F)
    WT2 = jnp.stack([W2e.T, W2c.T])
    S2 = jnp.stack([jnp.full((1, F), 2.0 + eps2e, jnp.float32),
                    jnp.full((1, F), 2.0 + eps2c, jnp.float32)])
    B2 = jnp.stack([b2e, b2c]).reshape(2, 1, F)

    src = edge_index[0]
    dst = edge_index[1]
    src_b = jnp.stack([src, src + N]).reshape(2, NSUB, NGROUPS, G, CHUNK)
    dst_b = jnp.broadcast_to(dst.reshape(1, NSUB, NGROUPS, G, CHUNK),
                             src_b.shape)
    sd_r = jnp.stack([src_b, dst_b], axis=4).reshape(
        2, NSUB, NGROUPS, SD, CHUNK)
    y1, base1 = _layer1_call(X, ME, WT1, S1, B1)
    acc1 = _segsum_call(y1.reshape(2 * N, F), base1.reshape(2 * N, F), sd_r)
    y2, base2 = _layer2_call(acc1.reshape(2, N, F), WT2, S2, B2)
    acc2 = _segsum_call(y2.reshape(2 * N, F), base2.reshape(2 * N, F), sd_r)

    maskf = jnp.broadcast_to(
        (central_node_index == 1).astype(jnp.float32)[:, None], (N, 8))
    xm, logits = _head_call(acc2.reshape(2, N, F), maskf, Wm.T,
                            bm.reshape(1, NC), Wmlp1.T, bmlp1.reshape(1, F),
                            Wmlp2.T, bmlp2.reshape(1, NC))
    return (xm, logits)
